# baseline (device time: 24286 ns/iter reference)
import jax
import jax.numpy as jnp
from jax import lax
from jax.experimental import pallas as pl
from jax.experimental.pallas import tpu as pltpu

N_DEV = 16

_FAR_FIRST = [8, 7, 9, 6, 10, 5, 11, 4, 12, 3, 13, 2, 14, 1, 15]


def kernel(t):
    m, n = t.shape
    mc = m // N_DEV

    def body(x_ref, out_ref, recv_ref, chunk_ref,
             send1_sems, recv1_sems, send2_sems, recv2_sems):
        my = lax.axis_index("i")

        barrier_sem = pltpu.get_barrier_semaphore()
        for j in range(N_DEV):
            @pl.when(my != j)
            def _():
                pl.semaphore_signal(
                    barrier_sem, inc=1,
                    device_id=(j,), device_id_type=pl.DeviceIdType.MESH,
                )
        pl.semaphore_wait(barrier_sem, N_DEV - 1)

        for s in _FAR_FIRST:
            j = lax.rem(my + s, N_DEV)
            rdma = pltpu.make_async_remote_copy(
                src_ref=x_ref.at[pl.ds(j * mc, mc)],
                dst_ref=recv_ref.at[my],
                send_sem=send1_sems.at[j],
                recv_sem=recv1_sems.at[my],
                device_id=(j,),
                device_id_type=pl.DeviceIdType.MESH,
            )
            rdma.start()

        recv_ref[my] = x_ref[pl.ds(my * mc, mc)]

        for k in range(N_DEV):
            @pl.when(my != k)
            def _():
                recv = pltpu.make_async_remote_copy(
                    src_ref=x_ref.at[pl.ds(0, mc)],
                    dst_ref=recv_ref.at[k],
                    send_sem=send1_sems.at[k],
                    recv_sem=recv1_sems.at[k],
                    device_id=(k,),
                    device_id_type=pl.DeviceIdType.MESH,
                )
                recv.wait_recv()

        sv = jnp.sum(recv_ref[...], axis=0)
        r = jnp.maximum(sv, 0.0)
        chunk_ref[...] = jnp.tanh(sv) * sv * sv + r * r * r

        for s in _FAR_FIRST:
            j = lax.rem(my + s, N_DEV)
            rdma = pltpu.make_async_remote_copy(
                src_ref=chunk_ref,
                dst_ref=out_ref.at[pl.ds(my * mc, mc)],
                send_sem=send2_sems.at[j],
                recv_sem=recv2_sems.at[my],
                device_id=(j,),
                device_id_type=pl.DeviceIdType.MESH,
            )
            rdma.start()

        out_ref[pl.ds(my * mc, mc)] = chunk_ref[...]

        for k in range(N_DEV):
            @pl.when(my != k)
            def _():
                recv = pltpu.make_async_remote_copy(
                    src_ref=chunk_ref,
                    dst_ref=out_ref.at[pl.ds(k * mc, mc)],
                    send_sem=send2_sems.at[k],
                    recv_sem=recv2_sems.at[k],
                    device_id=(k,),
                    device_id_type=pl.DeviceIdType.MESH,
                )
                recv.wait_recv()

        for j in range(N_DEV):
            @pl.when(my != j)
            def _():
                send = pltpu.make_async_remote_copy(
                    src_ref=chunk_ref,
                    dst_ref=out_ref.at[pl.ds(0, mc)],
                    send_sem=send2_sems.at[j],
                    recv_sem=recv2_sems.at[my],
                    device_id=(j,),
                    device_id_type=pl.DeviceIdType.MESH,
                )
                send.wait_send()
                send1 = pltpu.make_async_remote_copy(
                    src_ref=x_ref.at[pl.ds(j * mc, mc)],
                    dst_ref=recv_ref.at[my],
                    send_sem=send1_sems.at[j],
                    recv_sem=recv1_sems.at[my],
                    device_id=(j,),
                    device_id_type=pl.DeviceIdType.MESH,
                )
                send1.wait_send()

    return pl.pallas_call(
        body,
        out_shape=jax.ShapeDtypeStruct((m, n), jnp.float32),
        in_specs=[pl.BlockSpec(memory_space=pltpu.VMEM)],
        out_specs=pl.BlockSpec(memory_space=pltpu.VMEM),
        scratch_shapes=[
            pltpu.VMEM((N_DEV, mc, n), jnp.float32),
            pltpu.VMEM((mc, n), jnp.float32),
            pltpu.SemaphoreType.DMA((N_DEV,)),
            pltpu.SemaphoreType.DMA((N_DEV,)),
            pltpu.SemaphoreType.DMA((N_DEV,)),
            pltpu.SemaphoreType.DMA((N_DEV,)),
        ],
        compiler_params=pltpu.CompilerParams(collective_id=0),
    )(t)
